# initial kernel scaffold (unmeasured)
import jax
import jax.numpy as jnp
from jax import lax
from jax.experimental import pallas as pl
from jax.experimental.pallas import tpu as pltpu

N_DEV = 4


def kernel(x, w_mat):
    m_glob, k_shard = x.shape
    k_glob, n = w_mat.shape
    m_blk = m_glob // N_DEV
    k_blk = k_glob // N_DEV

    def body(x_hbm, w_hbm, out_ref, xstg, xb, comm, wstg, my_amax, amax_slots,
             xld_sems, wld_sems, send_sems, recv_sems, ax_send_sems,
             ax_recv_sems):
        my = lax.axis_index("i")

        bsem = pltpu.get_barrier_semaphore()
        for k in range(1, N_DEV):
            pl.semaphore_signal(bsem, inc=1, device_id=((my + k) % N_DEV,),
                                device_id_type=pl.DeviceIdType.MESH)
        pl.semaphore_wait(bsem, N_DEV - 1)

        def w_dma(src_off, slot):
            e = (my + src_off) % N_DEV
            c = pltpu.make_async_copy(w_hbm.at[pl.ds(e * k_blk, k_blk), :],
                                      wstg.at[slot], wld_sems.at[slot])
            c.start()
            return c

        wd_own = w_dma(0, 0)
        wd_first = w_dma(3, 1)

        def x_dma(k, slot):
            tgt = (my + k) % N_DEV
            c = pltpu.make_async_copy(x_hbm.at[pl.ds(tgt * m_blk, m_blk), :],
                                      xstg.at[slot], xld_sems.at[slot])
            c.start()
            return c

        order = [2, 1, 3, 0]
        dmas = {order[0]: x_dma(order[0], 0), order[1]: x_dma(order[1], 1)}
        send_rdmas = {}
        for i, k in enumerate(order):
            slot = i % 2
            dmas[k].wait()
            xb[k] = xstg[slot].astype(jnp.bfloat16)
            if k != 0:
                r = pltpu.make_async_remote_copy(
                    src_ref=xb.at[k],
                    dst_ref=comm.at[N_DEV - k],
                    send_sem=send_sems.at[k - 1],
                    recv_sem=recv_sems.at[N_DEV - k],
                    device_id=((my + k) % N_DEV,),
                    device_id_type=pl.DeviceIdType.MESH,
                )
                r.start()
                send_rdmas[k] = r
            if i + 2 < len(order):
                dmas[order[i + 2]] = x_dma(order[i + 2], slot)

        wd_own.wait()
        out_ref[...] = jnp.dot(xb[0], wstg[0].astype(jnp.bfloat16),
                               preferred_element_type=jnp.float32)
        wd_next = w_dma(1, 0)

        wd = {1: wd_first, 3: wd_next, 2: None}
        for k, wslot in ((1, 1), (3, 0), (2, 1)):
            send_rdmas[k].wait()
            wd[k].wait()
            if k == 3:
                wd[2] = w_dma(2, 1)
            out_ref[...] += jnp.dot(comm[N_DEV - k],
                                    wstg[wslot].astype(jnp.bfloat16),
                                    preferred_element_type=jnp.float32)

        local_amax = jnp.max(jnp.abs(out_ref[...]))
        my_amax[...] = jnp.full((8, 128), local_amax, dtype=jnp.float32)
        ax = []
        for k in range(1, N_DEV):
            r = pltpu.make_async_remote_copy(
                src_ref=my_amax,
                dst_ref=amax_slots.at[N_DEV - k],
                send_sem=ax_send_sems.at[k - 1],
                recv_sem=ax_recv_sems.at[N_DEV - k],
                device_id=((my + k) % N_DEV,),
                device_id_type=pl.DeviceIdType.MESH,
            )
            r.start()
            ax.append(r)
        for r in ax:
            r.wait()
        g = jnp.maximum(local_amax, jnp.max(amax_slots[1:N_DEV]))

        scale = g / 448.0
        q = (out_ref[...] / scale).astype(jnp.float8_e4m3fn)
        out_ref[...] = q.astype(jnp.float32) * scale

    return pl.pallas_call(
        body,
        out_shape=jax.ShapeDtypeStruct((m_blk, n), jnp.float32),
        in_specs=[
            pl.BlockSpec(memory_space=pltpu.ANY),
            pl.BlockSpec(memory_space=pltpu.ANY),
        ],
        out_specs=pl.BlockSpec(memory_space=pltpu.VMEM),
        scratch_shapes=[
            pltpu.VMEM((2, m_blk, k_shard), jnp.float32),
            pltpu.VMEM((N_DEV, m_blk, k_shard), jnp.bfloat16),
            pltpu.VMEM((N_DEV, m_blk, k_shard), jnp.bfloat16),
            pltpu.VMEM((2, k_blk, n), jnp.float32),
            pltpu.VMEM((8, 128), jnp.float32),
            pltpu.VMEM((N_DEV, 8, 128), jnp.float32),
            pltpu.SemaphoreType.DMA((2,)),
            pltpu.SemaphoreType.DMA((2,)),
            pltpu.SemaphoreType.DMA((3,)),
            pltpu.SemaphoreType.DMA((4,)),
            pltpu.SemaphoreType.DMA((3,)),
            pltpu.SemaphoreType.DMA((4,)),
        ],
        compiler_params=pltpu.CompilerParams(collective_id=0),
    )(x, w_mat)


# baseline (device time: 83740 ns/iter reference)
import jax
import jax.numpy as jnp
from jax import lax
from jax.experimental import pallas as pl
from jax.experimental.pallas import tpu as pltpu

N_DEV = 4


def kernel(x, w_mat):
    m_glob, k_shard = x.shape
    k_glob, n = w_mat.shape
    m_blk = m_glob // N_DEV
    k_blk = k_glob // N_DEV

    def body(x_hbm, w_hbm, out_ref, xstg, xb, comm, wstg, my_amax, amax_slots,
             xld_sems, wld_sems, send_sems, recv_sems, ax_send_sems,
             ax_recv_sems):
        my = lax.axis_index("i")

        bsem = pltpu.get_barrier_semaphore()
        for k in range(1, N_DEV):
            pl.semaphore_signal(bsem, inc=1, device_id=((my + k) % N_DEV,),
                                device_id_type=pl.DeviceIdType.MESH)
        pl.semaphore_wait(bsem, N_DEV - 1)

        def w_dma(src_off, slot):
            e = (my + src_off) % N_DEV
            c = pltpu.make_async_copy(w_hbm.at[pl.ds(e * k_blk, k_blk), :],
                                      wstg.at[slot], wld_sems.at[slot])
            c.start()
            return c

        wd_own = w_dma(0, 0)
        wd_first = w_dma(3, 1)

        def x_dma(k, slot):
            tgt = (my + k) % N_DEV
            c = pltpu.make_async_copy(x_hbm.at[pl.ds(tgt * m_blk, m_blk), :],
                                      xstg.at[slot], xld_sems.at[slot])
            c.start()
            return c

        order = [2, 1, 3, 0]
        dmas = {order[0]: x_dma(order[0], 0), order[1]: x_dma(order[1], 1)}
        send_rdmas = {}
        for i, k in enumerate(order):
            slot = i % 2
            dmas[k].wait()
            xb[k] = xstg[slot].astype(jnp.bfloat16)
            if k != 0:
                r = pltpu.make_async_remote_copy(
                    src_ref=xb.at[k],
                    dst_ref=comm.at[N_DEV - k],
                    send_sem=send_sems.at[k - 1],
                    recv_sem=recv_sems.at[N_DEV - k],
                    device_id=((my + k) % N_DEV,),
                    device_id_type=pl.DeviceIdType.MESH,
                )
                r.start()
                send_rdmas[k] = r
            if i + 2 < len(order):
                dmas[order[i + 2]] = x_dma(order[i + 2], slot)

        wd_own.wait()
        out_ref[...] = jnp.dot(xb[0], wstg[0].astype(jnp.bfloat16),
                               preferred_element_type=jnp.float32)
        wd_next = w_dma(1, 0)

        wd = {1: wd_first, 3: wd_next, 2: None}
        for k, wslot in ((1, 1), (3, 0), (2, 1)):
            send_rdmas[k].wait()
            wd[k].wait()
            if k == 3:
                wd[2] = w_dma(2, 1)
            out_ref[...] += jnp.dot(comm[N_DEV - k],
                                    wstg[wslot].astype(jnp.bfloat16),
                                    preferred_element_type=jnp.float32)

        local_amax = jnp.max(jnp.abs(out_ref[...]))
        my_amax[...] = jnp.full((8, 128), local_amax, dtype=jnp.float32)
        ax = []
        for k in range(1, N_DEV):
            r = pltpu.make_async_remote_copy(
                src_ref=my_amax,
                dst_ref=amax_slots.at[N_DEV - k],
                send_sem=ax_send_sems.at[k - 1],
                recv_sem=ax_recv_sems.at[N_DEV - k],
                device_id=((my + k) % N_DEV,),
                device_id_type=pl.DeviceIdType.MESH,
            )
            r.start()
            ax.append(r)
        for r in ax:
            r.wait()
        g = jnp.maximum(local_amax, jnp.max(amax_slots[1:N_DEV]))

        scale = g / 448.0
        q = (out_ref[...] / scale).astype(jnp.float8_e4m3fn)
        out_ref[...] = q.astype(jnp.float32) * scale

    return pl.pallas_call(
        body,
        out_shape=jax.ShapeDtypeStruct((m_blk, n), jnp.float32),
        in_specs=[
            pl.BlockSpec(memory_space=pl.ANY),
            pl.BlockSpec(memory_space=pl.ANY),
        ],
        out_specs=pl.BlockSpec(memory_space=pltpu.MemorySpace.VMEM),
        scratch_shapes=[
            pltpu.VMEM((2, m_blk, k_shard), jnp.float32),
            pltpu.VMEM((N_DEV, m_blk, k_shard), jnp.bfloat16),
            pltpu.VMEM((N_DEV, m_blk, k_shard), jnp.bfloat16),
            pltpu.VMEM((2, k_blk, n), jnp.float32),
            pltpu.VMEM((8, 128), jnp.float32),
            pltpu.VMEM((N_DEV, 8, 128), jnp.float32),
            pltpu.SemaphoreType.DMA((2,)),
            pltpu.SemaphoreType.DMA((2,)),
            pltpu.SemaphoreType.DMA((3,)),
            pltpu.SemaphoreType.DMA((4,)),
            pltpu.SemaphoreType.DMA((3,)),
            pltpu.SemaphoreType.DMA((4,)),
        ],
        compiler_params=pltpu.CompilerParams(
            collective_id=0,
            vmem_limit_bytes=63 * 1024 * 1024,
        ),
    )(x, w_mat)


# device time: 68168 ns/iter; 1.2284x vs baseline; 1.2284x over previous
import contextlib
import os

import jax
import jax.numpy as jnp
from jax import lax
from jax.experimental import pallas as pl
from jax.experimental.pallas import tpu as pltpu

N_DEV = 4
_SCOPES = os.environ.get("KERNEL_SCOPES", "0") == "1"
_MODE = os.environ.get("KERNEL_MODE", "full")


def _scope(name):
    return jax.named_scope(name) if _SCOPES else contextlib.nullcontext()


def kernel(x, w_mat):
    m_glob, k_shard = x.shape
    k_glob, n = w_mat.shape
    m_blk = m_glob // N_DEV
    k_blk = k_glob // N_DEV

    def body(x_hbm, w_hbm, out_ref, xstg, xb, comm, wstg, my_amax, amax_slots,
             xld_sems, wld_sems, send_sems, recv_sems, ax_send_sems,
             ax_recv_sems):
        my = lax.axis_index("i")

        with _scope("barrier"):
            bsem = pltpu.get_barrier_semaphore()
            for k in range(1, N_DEV):
                pl.semaphore_signal(bsem, inc=1,
                                    device_id=((my + k) % N_DEV,),
                                    device_id_type=pl.DeviceIdType.MESH)
            pl.semaphore_wait(bsem, N_DEV - 1)

        def w_dma(src_off, slot):
            e = (my + src_off) % N_DEV
            c = pltpu.make_async_copy(w_hbm.at[pl.ds(e * k_blk, k_blk), :],
                                      wstg.at[slot], wld_sems.at[slot])
            c.start()
            return c

        wd_own = w_dma(0, 0)
        wd_first = w_dma(3, 1)

        def x_dma(k, slot):
            tgt = (my + k) % N_DEV
            c = pltpu.make_async_copy(x_hbm.at[pl.ds(tgt * m_blk, m_blk), :],
                                      xstg.at[slot], xld_sems.at[slot])
            c.start()
            return c

        with _scope("xload_send"):
            order = [2, 1, 3, 0]
            dmas = {order[0]: x_dma(order[0], 0), order[1]: x_dma(order[1], 1)}
            send_rdmas = {}
            for i, k in enumerate(order):
                slot = i % 2
                dmas[k].wait()
                xb[k] = xstg[slot].astype(jnp.bfloat16)
                if k != 0:
                    r = pltpu.make_async_remote_copy(
                        src_ref=xb.at[k],
                        dst_ref=comm.at[N_DEV - k],
                        send_sem=send_sems.at[k - 1],
                        recv_sem=recv_sems.at[N_DEV - k],
                        device_id=((my + k) % N_DEV,),
                        device_id_type=pl.DeviceIdType.MESH,
                    )
                    r.start()
                    send_rdmas[k] = r
                if i + 2 < len(order):
                    dmas[order[i + 2]] = x_dma(order[i + 2], slot)

        with _scope("gemm_local"):
            wd_own.wait()
            if _MODE == "nogemm":
                out_ref[...] = jnp.zeros_like(out_ref)
            else:
                out_ref[...] = jnp.dot(xb[0], wstg[0].astype(jnp.bfloat16),
                                       preferred_element_type=jnp.float32)
            wd_next = w_dma(1, 0)

        wd = {1: wd_first, 3: wd_next, 2: None}
        for k, wslot in ((1, 1), (3, 0), (2, 1)):
            with _scope(f"wait_recv#k={k}"):
                send_rdmas[k].wait()
                wd[k].wait()
            with _scope(f"gemm#k={k}"):
                if k == 3:
                    wd[2] = w_dma(2, 1)
                if _MODE != "nogemm":
                    out_ref[...] += jnp.dot(comm[N_DEV - k],
                                            wstg[wslot].astype(jnp.bfloat16),
                                            preferred_element_type=jnp.float32)

        if _MODE in ("nogemm", "noepi"):
            return
        with _scope("amax"):
            local_amax = jnp.max(jnp.abs(out_ref[...]))
            my_amax[...] = jnp.full((8, 128), local_amax, dtype=jnp.float32)
            ax = []
            for k in range(1, N_DEV):
                r = pltpu.make_async_remote_copy(
                    src_ref=my_amax,
                    dst_ref=amax_slots.at[N_DEV - k],
                    send_sem=ax_send_sems.at[k - 1],
                    recv_sem=ax_recv_sems.at[N_DEV - k],
                    device_id=((my + k) % N_DEV,),
                    device_id_type=pl.DeviceIdType.MESH,
                )
                r.start()
                ax.append(r)
            for r in ax:
                r.wait()
            g = jnp.maximum(local_amax, jnp.max(amax_slots[1:N_DEV]))

        with _scope("quant"):
            scale = g / 448.0
            q = (out_ref[...] / scale).astype(jnp.float8_e4m3fn)
            out_ref[...] = q.astype(jnp.float32) * scale

    return pl.pallas_call(
        body,
        out_shape=jax.ShapeDtypeStruct((m_blk, n), jnp.float32),
        in_specs=[
            pl.BlockSpec(memory_space=pl.ANY),
            pl.BlockSpec(memory_space=pl.ANY),
        ],
        out_specs=pl.BlockSpec(memory_space=pltpu.MemorySpace.VMEM),
        scratch_shapes=[
            pltpu.VMEM((2, m_blk, k_shard), jnp.float32),
            pltpu.VMEM((N_DEV, m_blk, k_shard), jnp.bfloat16),
            pltpu.VMEM((N_DEV, m_blk, k_shard), jnp.bfloat16),
            pltpu.VMEM((2, k_blk, n), jnp.float32),
            pltpu.VMEM((8, 128), jnp.float32),
            pltpu.VMEM((N_DEV, 8, 128), jnp.float32),
            pltpu.SemaphoreType.DMA((2,)),
            pltpu.SemaphoreType.DMA((2,)),
            pltpu.SemaphoreType.DMA((3,)),
            pltpu.SemaphoreType.DMA((4,)),
            pltpu.SemaphoreType.DMA((3,)),
            pltpu.SemaphoreType.DMA((4,)),
        ],
        compiler_params=pltpu.CompilerParams(
            collective_id=0,
            vmem_limit_bytes=63 * 1024 * 1024,
        ),
    )(x, w_mat)
